# baseline (speedup 1.0000x reference)
import functools

import jax
import jax.numpy as jnp
from jax import lax
from jax.experimental import pallas as pl
from jax.experimental.pallas import tpu as pltpu
from jax.experimental.pallas import tpu_sc as plsc

_B = 4
_C = 96
_H = 224
_W = 224
_NE = 8
_CE = _C // _NE
_RB = 8
_RS = 96
_NRB = _RS // _RB
_NG = _W // 16
_TCH = 32


def _p1_body(in_hbm, t_hbm, si_hbm, st_hbm, ac_hbm,
             i0, t0, i1, t1, a_si, a_st, a_ac, b_si, b_st, b_ac,
             semi0, semi1, semo0, semo1):
    wid = lax.axis_index("s") * 2 + lax.axis_index("c")
    b = wid // _NE
    e = wid % _NE
    c0 = e * _CE

    def issue_in(rb, ib, tb, sem):
        pltpu.async_copy(
            in_hbm.at[b, pl.ds(c0, _CE), pl.ds(rb * _RB, _RB), :], ib, sem)
        pltpu.async_copy(
            t_hbm.at[b, pl.ds(c0, _CE), pl.ds(rb * _RB, _RB), :], tb, sem)

    def drain_in(rb, ib, tb, sem):
        pltpu.make_async_copy(
            in_hbm.at[b, pl.ds(c0, _CE), pl.ds(rb * _RB, _RB), :], ib,
            sem).wait()
        pltpu.make_async_copy(
            t_hbm.at[b, pl.ds(c0, _CE), pl.ds(rb * _RB, _RB), :], tb,
            sem).wait()

    def issue_out(rb, s_si, s_st, s_ac, sem):
        pltpu.async_copy(s_si, si_hbm.at[e, b, pl.ds(rb * _RB, _RB), :], sem)
        pltpu.async_copy(s_st, st_hbm.at[e, b, pl.ds(rb * _RB, _RB), :], sem)
        pltpu.async_copy(s_ac, ac_hbm.at[e, b, pl.ds(rb * _RB, _RB), :], sem)

    def drain_out(s_si, s_st, s_ac, sem):
        pltpu.make_async_copy(s_si, si_hbm.at[e, b, pl.ds(0, _RB), :],
                              sem).wait()
        pltpu.make_async_copy(s_st, st_hbm.at[e, b, pl.ds(0, _RB), :],
                              sem).wait()
        pltpu.make_async_copy(s_ac, ac_hbm.at[e, b, pl.ds(0, _RB), :],
                              sem).wait()

    zero = jnp.zeros((16,), jnp.float32)

    def compute(ib, tb, s_si, s_st, s_ac):
        def row_body(r, carry):
            @plsc.parallel_loop(0, _NG, step=1, unroll=2,
                                carry=jnp.int32(0))
            def grp_body(o, dummy):
                off = o * 16
                si = zero
                st = zero
                ac = zero
                for c in range(_CE):
                    iv = ib[c, r, pl.ds(off, 16)]
                    tv = tb[c, r, pl.ds(off, 16)]
                    te = jnp.exp(tv)
                    si = si + jnp.exp(iv)
                    st = st + te
                    ac = ac + te * (tv - iv)
                s_si[r, pl.ds(off, 16)] = si
                s_st[r, pl.ds(off, 16)] = st
                s_ac[r, pl.ds(off, 16)] = ac
                return dummy

            return carry

        lax.fori_loop(0, _RB, row_body, jnp.int32(0))

    issue_in(0, i0, t0, semi0)
    drain_in(0, i0, t0, semi0)
    issue_in(1, i1, t1, semi1)
    compute(i0, t0, a_si, a_st, a_ac)
    issue_out(0, a_si, a_st, a_ac, semo0)
    drain_in(1, i1, t1, semi1)
    issue_in(2, i0, t0, semi0)
    compute(i1, t1, b_si, b_st, b_ac)
    issue_out(1, b_si, b_st, b_ac, semo1)

    def pair_body(k, carry):
        rb_a = 2 * k
        rb_b = rb_a + 1
        rb_n = jnp.minimum(rb_a + 2, _NRB - 1)
        drain_in(rb_a, i0, t0, semi0)
        issue_in(rb_b, i1, t1, semi1)
        drain_out(a_si, a_st, a_ac, semo0)
        compute(i0, t0, a_si, a_st, a_ac)
        issue_out(rb_a, a_si, a_st, a_ac, semo0)
        drain_in(rb_b, i1, t1, semi1)
        issue_in(rb_n, i0, t0, semi0)
        drain_out(b_si, b_st, b_ac, semo1)
        compute(i1, t1, b_si, b_st, b_ac)
        issue_out(rb_b, b_si, b_st, b_ac, semo1)
        return carry

    lax.fori_loop(1, _NRB // 2, pair_body, jnp.int32(0))
    drain_in(_NRB - 1, i0, t0, semi0)
    drain_out(a_si, a_st, a_ac, semo0)
    drain_out(b_si, b_st, b_ac, semo1)


def _tc_body(in_ref, t_ref, lab_ref, s_ref, n_ref):
    i = pl.program_id(0)
    j = pl.program_id(1)

    @pl.when((i == 0) & (j == 0))
    def _():
        s_ref[...] = jnp.zeros_like(s_ref)
        n_ref[...] = jnp.zeros_like(n_ref)

    iv = in_ref[0]
    tv = t_ref[0]
    te = jnp.exp(tv)
    ei = jnp.exp(iv)
    pr = te * (tv - iv)
    ones = jnp.ones((1, _C), jnp.float32)
    dn = (((1,), (0,)), ((), ()))
    s_tot = jnp.zeros((), jnp.float32)
    n_tot = jnp.zeros((), jnp.float32)
    for r in range(_TCH):
        si = lax.dot_general(ones, ei[:, r, :], dn,
                             preferred_element_type=jnp.float32)
        st = lax.dot_general(ones, te[:, r, :], dn,
                             preferred_element_type=jnp.float32)
        ac = lax.dot_general(ones, pr[:, r, :], dn,
                             preferred_element_type=jnp.float32)
        kl = ac / st + jnp.log(si) - jnp.log(st)
        m = lab_ref[0, r, :][None, :] != 0
        s_tot += jnp.sum(jnp.where(m, kl, 0.0))
        n_tot += jnp.sum(jnp.where(m, 1.0, 0.0))
    s_ref[...] += s_tot[None, None]
    n_ref[...] += n_tot[None, None]


def _p2_body(si_ref, st_ref, ac_ref, lab_ref, ts_ref, tn_ref,
             s_ref, n_ref, l_ref):
    i = pl.program_id(0)
    j = pl.program_id(1)

    @pl.when((i == 0) & (j == 0))
    def _():
        s_ref[...] = jnp.zeros_like(s_ref)
        n_ref[...] = jnp.zeros_like(n_ref)

    si = jnp.sum(si_ref[...], axis=0)
    st = jnp.sum(st_ref[...], axis=0)
    ac = jnp.sum(ac_ref[...], axis=0)
    kl = ac / st + jnp.log(si) - jnp.log(st)
    m = lab_ref[...] != 0
    s_ref[...] += jnp.sum(jnp.where(m, kl, 0.0))[None, None]
    n_ref[...] += jnp.sum(jnp.where(m, 1.0, 0.0))[None, None]

    @pl.when((i == _B - 1) & (j == pl.num_programs(1) - 1))
    def _():
        l_ref[...] = ((s_ref[...] + ts_ref[...])
                      / (n_ref[...] + tn_ref[...]))


def kernel(input, target, label):
    lab = label.astype(jnp.int32)

    mesh = plsc.VectorSubcoreMesh(core_axis_name="c", subcore_axis_name="s")
    p1 = functools.partial(
        pl.kernel,
        mesh=mesh,
        out_type=[
            jax.ShapeDtypeStruct((_NE, _B, _RS, _W), jnp.float32),
            jax.ShapeDtypeStruct((_NE, _B, _RS, _W), jnp.float32),
            jax.ShapeDtypeStruct((_NE, _B, _RS, _W), jnp.float32),
        ],
        scratch_types=[
            pltpu.VMEM((_CE, _RB, _W), jnp.float32),
            pltpu.VMEM((_CE, _RB, _W), jnp.float32),
            pltpu.VMEM((_CE, _RB, _W), jnp.float32),
            pltpu.VMEM((_CE, _RB, _W), jnp.float32),
            pltpu.VMEM((_RB, _W), jnp.float32),
            pltpu.VMEM((_RB, _W), jnp.float32),
            pltpu.VMEM((_RB, _W), jnp.float32),
            pltpu.VMEM((_RB, _W), jnp.float32),
            pltpu.VMEM((_RB, _W), jnp.float32),
            pltpu.VMEM((_RB, _W), jnp.float32),
            pltpu.SemaphoreType.DMA,
            pltpu.SemaphoreType.DMA,
            pltpu.SemaphoreType.DMA,
            pltpu.SemaphoreType.DMA,
        ],
    )(_p1_body)
    si_p, st_p, ac_p = p1(input, target)

    nb = (_H - _RS) // _TCH
    ts2d, tn2d = pl.pallas_call(
        _tc_body,
        grid=(_B, nb),
        in_specs=[
            pl.BlockSpec((1, _C, _TCH, _W),
                         lambda i, j: (i, 0, _RS // _TCH + j, 0)),
            pl.BlockSpec((1, _C, _TCH, _W),
                         lambda i, j: (i, 0, _RS // _TCH + j, 0)),
            pl.BlockSpec((1, _TCH, _W), lambda i, j: (i, _RS // _TCH + j, 0)),
        ],
        out_specs=[
            pl.BlockSpec((1, 1), lambda i, j: (0, 0)),
            pl.BlockSpec((1, 1), lambda i, j: (0, 0)),
        ],
        out_shape=[
            jax.ShapeDtypeStruct((1, 1), jnp.float32),
            jax.ShapeDtypeStruct((1, 1), jnp.float32),
        ],
    )(input, target, lab)

    rows = 48
    grid = (_B, _RS // rows)
    _, _, loss2d = pl.pallas_call(
        _p2_body,
        grid=grid,
        in_specs=[
            pl.BlockSpec((_NE, 1, rows, _W), lambda i, j: (0, i, j, 0)),
            pl.BlockSpec((_NE, 1, rows, _W), lambda i, j: (0, i, j, 0)),
            pl.BlockSpec((_NE, 1, rows, _W), lambda i, j: (0, i, j, 0)),
            pl.BlockSpec((1, rows, _W), lambda i, j: (i, j, 0)),
            pl.BlockSpec((1, 1), lambda i, j: (0, 0)),
            pl.BlockSpec((1, 1), lambda i, j: (0, 0)),
        ],
        out_specs=[
            pl.BlockSpec((1, 1), lambda i, j: (0, 0)),
            pl.BlockSpec((1, 1), lambda i, j: (0, 0)),
            pl.BlockSpec((1, 1), lambda i, j: (0, 0)),
        ],
        out_shape=[
            jax.ShapeDtypeStruct((1, 1), jnp.float32),
            jax.ShapeDtypeStruct((1, 1), jnp.float32),
            jax.ShapeDtypeStruct((1, 1), jnp.float32),
        ],
    )(si_p, st_p, ac_p, lab, ts2d, tn2d)
    return loss2d[0, 0]


# speedup vs baseline: 1.4267x; 1.4267x over previous
import functools

import jax
import jax.numpy as jnp
from jax import lax
from jax.experimental import pallas as pl
from jax.experimental.pallas import tpu as pltpu
from jax.experimental.pallas import tpu_sc as plsc

_B = 4
_C = 96
_H = 224
_W = 224
_NE = 8
_CE = _C // _NE
_RB = 8
_RS = 112
_NRB = _RS // _RB
_NG = _W // 16
_TCH = 56


def _p1_body(in_hbm, t_hbm, si_hbm, st_hbm, ac_hbm,
             i0, t0, i1, t1, a_si, a_st, a_ac, b_si, b_st, b_ac,
             semi0, semi1, semo0, semo1):
    wid = lax.axis_index("s") * 2 + lax.axis_index("c")
    b = wid // _NE
    e = wid % _NE
    c0 = e * _CE

    def issue_in(rb, ib, tb, sem):
        pltpu.async_copy(
            in_hbm.at[b, pl.ds(c0, _CE), pl.ds(rb * _RB, _RB), :], ib, sem)
        pltpu.async_copy(
            t_hbm.at[b, pl.ds(c0, _CE), pl.ds(rb * _RB, _RB), :], tb, sem)

    def drain_in(rb, ib, tb, sem):
        pltpu.make_async_copy(
            in_hbm.at[b, pl.ds(c0, _CE), pl.ds(rb * _RB, _RB), :], ib,
            sem).wait()
        pltpu.make_async_copy(
            t_hbm.at[b, pl.ds(c0, _CE), pl.ds(rb * _RB, _RB), :], tb,
            sem).wait()

    def issue_out(rb, s_si, s_st, s_ac, sem):
        pltpu.async_copy(s_si, si_hbm.at[e, b, pl.ds(rb * _RB, _RB), :], sem)
        pltpu.async_copy(s_st, st_hbm.at[e, b, pl.ds(rb * _RB, _RB), :], sem)
        pltpu.async_copy(s_ac, ac_hbm.at[e, b, pl.ds(rb * _RB, _RB), :], sem)

    def drain_out(s_si, s_st, s_ac, sem):
        pltpu.make_async_copy(s_si, si_hbm.at[e, b, pl.ds(0, _RB), :],
                              sem).wait()
        pltpu.make_async_copy(s_st, st_hbm.at[e, b, pl.ds(0, _RB), :],
                              sem).wait()
        pltpu.make_async_copy(s_ac, ac_hbm.at[e, b, pl.ds(0, _RB), :],
                              sem).wait()

    zero = jnp.zeros((16,), jnp.float32)

    def compute(ib, tb, s_si, s_st, s_ac):
        def row_body(r, carry):
            @plsc.parallel_loop(0, _NG, step=1, unroll=7,
                                carry=jnp.int32(0))
            def grp_body(o, dummy):
                off = o * 16
                si = zero
                st = zero
                ac = zero
                for c in range(_CE):
                    iv = ib[c, r, pl.ds(off, 16)]
                    tv = tb[c, r, pl.ds(off, 16)]
                    te = jnp.exp(tv)
                    si = si + jnp.exp(iv)
                    st = st + te
                    ac = ac + te * (tv - iv)
                s_si[r, pl.ds(off, 16)] = si
                s_st[r, pl.ds(off, 16)] = st
                s_ac[r, pl.ds(off, 16)] = ac
                return dummy

            return carry

        lax.fori_loop(0, _RB, row_body, jnp.int32(0))

    issue_in(0, i0, t0, semi0)
    drain_in(0, i0, t0, semi0)
    issue_in(1, i1, t1, semi1)
    compute(i0, t0, a_si, a_st, a_ac)
    issue_out(0, a_si, a_st, a_ac, semo0)
    drain_in(1, i1, t1, semi1)
    issue_in(2, i0, t0, semi0)
    compute(i1, t1, b_si, b_st, b_ac)
    issue_out(1, b_si, b_st, b_ac, semo1)

    def pair_body(k, carry):
        rb_a = 2 * k
        rb_b = rb_a + 1
        rb_n = jnp.minimum(rb_a + 2, _NRB - 1)
        drain_in(rb_a, i0, t0, semi0)
        issue_in(rb_b, i1, t1, semi1)
        drain_out(a_si, a_st, a_ac, semo0)
        compute(i0, t0, a_si, a_st, a_ac)
        issue_out(rb_a, a_si, a_st, a_ac, semo0)
        drain_in(rb_b, i1, t1, semi1)
        issue_in(rb_n, i0, t0, semi0)
        drain_out(b_si, b_st, b_ac, semo1)
        compute(i1, t1, b_si, b_st, b_ac)
        issue_out(rb_b, b_si, b_st, b_ac, semo1)
        return carry

    lax.fori_loop(1, _NRB // 2, pair_body, jnp.int32(0))
    drain_in(_NRB - 1, i0, t0, semi0)
    drain_out(a_si, a_st, a_ac, semo0)
    drain_out(b_si, b_st, b_ac, semo1)


def _tc_body(in_ref, t_ref, lab_ref, s_ref, n_ref):
    i = pl.program_id(0)
    j = pl.program_id(1)

    @pl.when((i == 0) & (j == 0))
    def _():
        s_ref[...] = jnp.zeros_like(s_ref)
        n_ref[...] = jnp.zeros_like(n_ref)

    iv = in_ref[0]
    tv = t_ref[0]
    te = jnp.exp(tv)
    si = jnp.sum(jnp.exp(iv), axis=0)
    st = jnp.sum(te, axis=0)
    ac = jnp.sum(te * (tv - iv), axis=0)
    kl = ac / st + jnp.log(si) - jnp.log(st)
    m = lab_ref[0] != 0
    s_ref[...] += jnp.sum(jnp.where(m, kl, 0.0))[None, None]
    n_ref[...] += jnp.sum(jnp.where(m, 1.0, 0.0))[None, None]


def _p2_body(si_ref, st_ref, ac_ref, lab_ref, ts_ref, tn_ref,
             s_ref, n_ref, l_ref):
    i = pl.program_id(0)
    j = pl.program_id(1)

    @pl.when((i == 0) & (j == 0))
    def _():
        s_ref[...] = jnp.zeros_like(s_ref)
        n_ref[...] = jnp.zeros_like(n_ref)

    si = jnp.sum(si_ref[...], axis=0)
    st = jnp.sum(st_ref[...], axis=0)
    ac = jnp.sum(ac_ref[...], axis=0)
    kl = ac / st + jnp.log(si) - jnp.log(st)
    m = lab_ref[...] != 0
    s_ref[...] += jnp.sum(jnp.where(m, kl, 0.0))[None, None]
    n_ref[...] += jnp.sum(jnp.where(m, 1.0, 0.0))[None, None]

    @pl.when((i == _B - 1) & (j == pl.num_programs(1) - 1))
    def _():
        l_ref[...] = ((s_ref[...] + ts_ref[...])
                      / (n_ref[...] + tn_ref[...]))


def kernel(input, target, label):
    lab = label.astype(jnp.int32)

    mesh = plsc.VectorSubcoreMesh(core_axis_name="c", subcore_axis_name="s")
    p1 = functools.partial(
        pl.kernel,
        mesh=mesh,
        out_type=[
            jax.ShapeDtypeStruct((_NE, _B, _RS, _W), jnp.float32),
            jax.ShapeDtypeStruct((_NE, _B, _RS, _W), jnp.float32),
            jax.ShapeDtypeStruct((_NE, _B, _RS, _W), jnp.float32),
        ],
        scratch_types=[
            pltpu.VMEM((_CE, _RB, _W), jnp.float32),
            pltpu.VMEM((_CE, _RB, _W), jnp.float32),
            pltpu.VMEM((_CE, _RB, _W), jnp.float32),
            pltpu.VMEM((_CE, _RB, _W), jnp.float32),
            pltpu.VMEM((_RB, _W), jnp.float32),
            pltpu.VMEM((_RB, _W), jnp.float32),
            pltpu.VMEM((_RB, _W), jnp.float32),
            pltpu.VMEM((_RB, _W), jnp.float32),
            pltpu.VMEM((_RB, _W), jnp.float32),
            pltpu.VMEM((_RB, _W), jnp.float32),
            pltpu.SemaphoreType.DMA,
            pltpu.SemaphoreType.DMA,
            pltpu.SemaphoreType.DMA,
            pltpu.SemaphoreType.DMA,
        ],
    )(_p1_body)
    si_p, st_p, ac_p = p1(input, target)

    nb = (_H - _RS) // _TCH
    ts2d, tn2d = pl.pallas_call(
        _tc_body,
        grid=(_B, nb),
        in_specs=[
            pl.BlockSpec((1, _C, _TCH, _W),
                         lambda i, j: (i, 0, _RS // _TCH + j, 0)),
            pl.BlockSpec((1, _C, _TCH, _W),
                         lambda i, j: (i, 0, _RS // _TCH + j, 0)),
            pl.BlockSpec((1, _TCH, _W), lambda i, j: (i, _RS // _TCH + j, 0)),
        ],
        out_specs=[
            pl.BlockSpec((1, 1), lambda i, j: (0, 0)),
            pl.BlockSpec((1, 1), lambda i, j: (0, 0)),
        ],
        out_shape=[
            jax.ShapeDtypeStruct((1, 1), jnp.float32),
            jax.ShapeDtypeStruct((1, 1), jnp.float32),
        ],
    )(input, target, lab)

    rows = 56
    grid = (_B, _RS // rows)
    _, _, loss2d = pl.pallas_call(
        _p2_body,
        grid=grid,
        in_specs=[
            pl.BlockSpec((_NE, 1, rows, _W), lambda i, j: (0, i, j, 0)),
            pl.BlockSpec((_NE, 1, rows, _W), lambda i, j: (0, i, j, 0)),
            pl.BlockSpec((_NE, 1, rows, _W), lambda i, j: (0, i, j, 0)),
            pl.BlockSpec((1, rows, _W), lambda i, j: (i, j, 0)),
            pl.BlockSpec((1, 1), lambda i, j: (0, 0)),
            pl.BlockSpec((1, 1), lambda i, j: (0, 0)),
        ],
        out_specs=[
            pl.BlockSpec((1, 1), lambda i, j: (0, 0)),
            pl.BlockSpec((1, 1), lambda i, j: (0, 0)),
            pl.BlockSpec((1, 1), lambda i, j: (0, 0)),
        ],
        out_shape=[
            jax.ShapeDtypeStruct((1, 1), jnp.float32),
            jax.ShapeDtypeStruct((1, 1), jnp.float32),
            jax.ShapeDtypeStruct((1, 1), jnp.float32),
        ],
    )(si_p, st_p, ac_p, lab, ts2d, tn2d)
    return loss2d[0, 0]
